# baseline (speedup 1.0000x reference)
import functools

import jax
import jax.numpy as jnp
from jax import lax
from jax.experimental import pallas as pl
from jax.experimental.pallas import tpu as pltpu
from jax.experimental.pallas import tpu_sc as plsc


def _sc_posadd(x_flat, pos_flat, B, S, H):
    NC, NS, L = 2, 16, 16
    NW = NC * NS
    seq_per_w = S // NW
    C = 16
    nsub = seq_per_w // C
    mesh = plsc.VectorSubcoreMesh(
        core_axis_name="c", subcore_axis_name="s", num_cores=NC)

    @functools.partial(
        pl.kernel,
        out_type=jax.ShapeDtypeStruct((B * S * H,), jnp.float32),
        mesh=mesh,
        scratch_types=[
            pltpu.VMEM((C * H,), jnp.float32),
            pltpu.VMEM((C * H,), jnp.float32),
        ],
    )
    def k(x_hbm, pos_hbm, out_hbm, bufp, bufx):
        wid = lax.axis_index("s") * NC + lax.axis_index("c")
        s0 = wid * seq_per_w
        for c in range(nsub):
            row = s0 + c * C
            pltpu.sync_copy(pos_hbm.at[pl.ds(row * H, C * H)], bufp)
            for b in range(B):
                off = (b * S + row) * H
                pltpu.sync_copy(x_hbm.at[pl.ds(off, C * H)], bufx)

                def body(j, carry):
                    o = j * L
                    bufx[pl.ds(o, L)] = bufx[pl.ds(o, L)] + bufp[pl.ds(o, L)]
                    return carry

                lax.fori_loop(0, C * H // L, body, 0)
                pltpu.sync_copy(bufx, out_hbm.at[pl.ds(off, C * H)])

    return k(x_flat, pos_flat)


def kernel(x, pos_table):
    B, S, H = x.shape
    out_flat = _sc_posadd(x.reshape(-1), pos_table.reshape(-1), B, S, H)
    return out_flat.reshape(B, S, H)


# speedup vs baseline: 1.7178x; 1.7178x over previous
import functools

import jax
import jax.numpy as jnp
from jax import lax
from jax.experimental import pallas as pl
from jax.experimental.pallas import tpu as pltpu
from jax.experimental.pallas import tpu_sc as plsc


def _sc_posadd(x_flat, pos_flat, B, S, H):
    NC, NS, L = 2, 16, 16
    NW = NC * NS
    seq_per_w = S // NW
    C = 16
    nsub = seq_per_w // C
    T = nsub * B
    NBUF = 3
    U = 8
    mesh = plsc.VectorSubcoreMesh(
        core_axis_name="c", subcore_axis_name="s", num_cores=NC)

    @functools.partial(
        pl.kernel,
        out_type=jax.ShapeDtypeStruct((B * S * H,), jnp.float32),
        mesh=mesh,
        scratch_types=(
            [pltpu.VMEM((C * H,), jnp.float32) for _ in range(2)]
            + [pltpu.VMEM((C * H,), jnp.float32) for _ in range(NBUF)]
            + [pltpu.SemaphoreType.DMA for _ in range(2 + 2 * NBUF)]
        ),
    )
    def k(x_hbm, pos_hbm, out_hbm, bp0, bp1, bx0, bx1, bx2,
          sp0, sp1, sx0, sx1, sx2, ss0, ss1, ss2):
        bufp, semp = [bp0, bp1], [sp0, sp1]
        bufx, semx = [bx0, bx1, bx2], [sx0, sx1, sx2]
        sems = [ss0, ss1, ss2]
        wid = lax.axis_index("s") * NC + lax.axis_index("c")
        s0 = wid * seq_per_w

        def x_off(t):
            c, b = divmod(t, B)
            return (b * S + s0 + c * C) * H

        def start_in(t):
            return pltpu.async_copy(
                x_hbm.at[pl.ds(x_off(t), C * H)], bufx[t % NBUF],
                semx[t % NBUF])

        def start_pos(c):
            return pltpu.async_copy(
                pos_hbm.at[pl.ds((s0 + c * C) * H, C * H)], bufp[c % 2],
                semp[c % 2])

        pos_copy = [None] * nsub
        in_copy = [None] * T
        st_copy = [None] * T
        pos_copy[0] = start_pos(0)
        if nsub > 1:
            pos_copy[1] = start_pos(1)
        in_copy[0] = start_in(0)

        for t in range(T):
            c, b = divmod(t, B)
            if t + 1 < T:
                if t + 1 - NBUF >= 0:
                    st_copy[t + 1 - NBUF].wait()
                in_copy[t + 1] = start_in(t + 1)
            if b == 0:
                pos_copy[c].wait()
            in_copy[t].wait()

            bx, bp = bufx[t % NBUF], bufp[c % 2]

            @plsc.parallel_loop(0, C * H, step=L, unroll=U)
            def _(o):
                bx[pl.ds(o, L)] = bx[pl.ds(o, L)] + bp[pl.ds(o, L)]

            st_copy[t] = pltpu.async_copy(
                bx, out_hbm.at[pl.ds(x_off(t), C * H)], sems[t % NBUF])
            if b == B - 1 and c + 2 < nsub:
                pos_copy[c + 2] = start_pos(c + 2)

        for t in range(max(0, T - NBUF), T):
            st_copy[t].wait()

    return k(x_flat, pos_flat)


def kernel(x, pos_table):
    B, S, H = x.shape
    out_flat = _sc_posadd(x.reshape(-1), pos_table.reshape(-1), B, S, H)
    return out_flat.reshape(B, S, H)


# speedup vs baseline: 3.3145x; 1.9294x over previous
import functools

import jax
import jax.numpy as jnp
from jax import lax
from jax.experimental import pallas as pl
from jax.experimental.pallas import tpu as pltpu
from jax.experimental.pallas import tpu_sc as plsc


def _sc_posadd(x, pos_table, B, S, H):
    NC, NS, L = 2, 16, 16
    NW = NC * NS
    seq_per_w = S // NW
    C = 16
    nsub = seq_per_w // C
    T = nsub * B
    NBUF = 3
    U = 2
    mesh = plsc.VectorSubcoreMesh(
        core_axis_name="c", subcore_axis_name="s", num_cores=NC)

    @functools.partial(
        pl.kernel,
        out_type=jax.ShapeDtypeStruct((B, S, H), jnp.float32),
        mesh=mesh,
        scratch_types=(
            [pltpu.VMEM((C, H), jnp.float32) for _ in range(2)]
            + [pltpu.VMEM((C, H), jnp.float32) for _ in range(NBUF)]
            + [pltpu.SemaphoreType.DMA for _ in range(2 + 2 * NBUF)]
        ),
    )
    def k(x_hbm, pos_hbm, out_hbm, bp0, bp1, bx0, bx1, bx2,
          sp0, sp1, sx0, sx1, sx2, ss0, ss1, ss2):
        bufp, semp = [bp0, bp1], [sp0, sp1]
        bufx, semx = [bx0, bx1, bx2], [sx0, sx1, sx2]
        sems = [ss0, ss1, ss2]
        wid = lax.axis_index("s") * NC + lax.axis_index("c")
        s0 = wid * seq_per_w

        def start_in(t):
            c, b = divmod(t, B)
            return pltpu.async_copy(
                x_hbm.at[b, pl.ds(s0 + c * C, C)], bufx[t % NBUF],
                semx[t % NBUF])

        def start_pos(c):
            return pltpu.async_copy(
                pos_hbm.at[pl.ds(s0 + c * C, C)], bufp[c % 2], semp[c % 2])

        pos_copy = [None] * nsub
        in_copy = [None] * T
        st_copy = [None] * T
        pos_copy[0] = start_pos(0)
        if nsub > 1:
            pos_copy[1] = start_pos(1)
        in_copy[0] = start_in(0)

        for t in range(T):
            c, b = divmod(t, B)
            if t + 1 < T:
                if t + 1 - NBUF >= 0:
                    st_copy[t + 1 - NBUF].wait()
                in_copy[t + 1] = start_in(t + 1)
            if b == 0:
                pos_copy[c].wait()
            in_copy[t].wait()

            bx, bp = bufx[t % NBUF], bufp[c % 2]

            @plsc.parallel_loop(0, H, step=L, unroll=U)
            def _(o):
                for r in range(C):
                    bx[r, pl.ds(o, L)] = bx[r, pl.ds(o, L)] + bp[r, pl.ds(o, L)]

            st_copy[t] = pltpu.async_copy(
                bx, out_hbm.at[b, pl.ds(s0 + c * C, C)], sems[t % NBUF])
            if b == B - 1 and c + 2 < nsub:
                pos_copy[c + 2] = start_pos(c + 2)

        for t in range(max(0, T - NBUF), T):
            st_copy[t].wait()

    return k(x, pos_table)


def kernel(x, pos_table):
    B, S, H = x.shape
    return _sc_posadd(x, pos_table, B, S, H)


# speedup vs baseline: 3.4425x; 1.0386x over previous
import functools

import jax
import jax.numpy as jnp
from jax import lax
from jax.experimental import pallas as pl
from jax.experimental.pallas import tpu as pltpu
from jax.experimental.pallas import tpu_sc as plsc


def _sc_posadd(x2, pos_table, B, S, H):
    NC, NS, L = 2, 16, 16
    NW = NC * NS
    seq_per_w = S // NW
    C = 8
    nsub = seq_per_w // C
    NBUF = 4
    STEP_BYTES = C * H * 4
    mesh = plsc.VectorSubcoreMesh(
        core_axis_name="c", subcore_axis_name="s", num_cores=NC)

    @functools.partial(
        pl.kernel,
        out_type=jax.ShapeDtypeStruct((B * S, H), jnp.float32),
        mesh=mesh,
        scratch_types=(
            [pltpu.VMEM((seq_per_w, H), jnp.float32)]
            + [pltpu.VMEM((C, H), jnp.float32) for _ in range(NBUF)]
            + [pltpu.SemaphoreType.DMA for _ in range(1 + 2 * NBUF)]
        ),
    )
    def k(x_hbm, pos_hbm, out_hbm, bufpos, bx0, bx1, bx2, bx3,
          sempos, sx0, sx1, sx2, sx3, ss0, ss1, ss2, ss3):
        bufx = [bx0, bx1, bx2, bx3]
        semx = [sx0, sx1, sx2, sx3]
        sems = [ss0, ss1, ss2, ss3]
        wid = lax.axis_index("s") * NC + lax.axis_index("c")
        s0 = wid * seq_per_w

        pos_copy = pltpu.async_copy(
            pos_hbm.at[pl.ds(s0, seq_per_w)], bufpos, sempos)
        pltpu.async_copy(x_hbm.at[pl.ds(s0, C)], bufx[0], semx[0])
        pos_copy.wait()

        def run_batch(g, first):
            for i in range(nsub):
                j, jn = i % NBUF, (i + 1) % NBUF
                if not (first and i + 1 < NBUF):
                    pltpu.make_async_copy(
                        bufx[jn], out_hbm.at[pl.ds(s0, C)], sems[jn]).wait()
                if i + 1 < nsub:
                    nrow = g * S + s0 + (i + 1) * C
                else:
                    nrow = jnp.minimum(g + 1, B - 1) * S + s0
                pltpu.async_copy(x_hbm.at[pl.ds(nrow, C)], bufx[jn], semx[jn])
                pltpu.make_async_copy(
                    x_hbm.at[pl.ds(s0, C)], bufx[j], semx[j]).wait()

                bx = bufx[j]

                @plsc.parallel_loop(0, H, step=L, unroll=2)
                def _(o):
                    for r in range(C):
                        plsc.addupdate(
                            bx.at[r, pl.ds(o, L)],
                            bufpos[i * C + r, pl.ds(o, L)])

                pltpu.async_copy(
                    bx, out_hbm.at[pl.ds(g * S + s0 + i * C, C)], sems[j])

        run_batch(0, True)

        def body(g, carry):
            run_batch(g, False)
            return carry

        lax.fori_loop(1, B, body, 0)

        for j in range(1, NBUF):
            pltpu.make_async_copy(
                bufx[j], out_hbm.at[pl.ds(s0, C)], sems[j]).wait()
        pltpu.make_async_copy(
            x_hbm.at[pl.ds(s0, C)], bufx[0], semx[0]).wait()

    return k(x2, pos_table)


def kernel(x, pos_table):
    B, S, H = x.shape
    out2 = _sc_posadd(x.reshape(B * S, H), pos_table, B, S, H)
    return out2.reshape(B, S, H)
